# baseline (speedup 1.0000x reference)
import functools

import jax
import jax.numpy as jnp
from jax import lax
from jax.experimental import pallas as pl
from jax.experimental.pallas import tpu as pltpu
from jax.experimental.pallas import tpu_sc as plsc

N = 10000
E = 320000
D_IN = 128
D_HID = 128
D_OUT = 16

NC = 2
NS = 16
NW = NC * NS
CHUNK = 80
EPW = E // NW
NCHUNK = EPW // CHUNK
SB = 25
NSTAGE = NCHUNK // SB
NP = 10240
RPS = NP // NS
BR = 1024


def _mesh():
    return plsc.VectorSubcoreMesh(core_axis_name="c", subcore_axis_name="s")


def _sc_deg_body(dst_hbm, ew_hbm, zeros_hbm, degp_hbm, dstbuf, ewbuf, msg, acc):
    c = lax.axis_index("c")
    s = lax.axis_index("s")
    wid = c * NS + s
    pltpu.sync_copy(dst_hbm.at[wid], dstbuf)
    pltpu.sync_copy(ew_hbm.at[wid], ewbuf)
    nb = s * RPS
    pltpu.sync_copy(zeros_hbm.at[pl.ds(nb, RPS)], acc.at[pl.ds(nb, RPS)])
    lane = lax.iota(jnp.int32, 16)
    base_row = jnp.where(lane == 0, 1.0, 0.0).astype(jnp.float32)
    e1 = jnp.where(lane == 1, 1.0, 0.0).astype(jnp.float32)
    plsc.subcore_barrier()

    def body(j, carry):
        for g in range(CHUNK // 16):
            ewv = ewbuf[j, pl.ds(g * 16, 16)]
            for k in range(16):
                msg[g * 16 + k] = base_row + e1 * ewv[k]
        pltpu.sync_copy(msg, acc.at[dstbuf.at[j]], add=True)
        return carry

    lax.fori_loop(0, NCHUNK, body, 0)
    plsc.subcore_barrier()
    pltpu.sync_copy(acc.at[pl.ds(nb, RPS)], degp_hbm.at[c, pl.ds(nb, RPS)])


def _sc_deg(dst2d, ew2d, z16):
    fn = functools.partial(
        pl.kernel,
        out_type=jax.ShapeDtypeStruct((NC, NP, 16), jnp.float32),
        mesh=_mesh(),
        compiler_params=pltpu.CompilerParams(use_tc_tiling_on_sc=False),
        scratch_types=[
            pltpu.VMEM((NCHUNK, CHUNK), jnp.int32),
            pltpu.VMEM((NCHUNK, CHUNK), jnp.float32),
            pltpu.VMEM((CHUNK, 16), jnp.float32),
            pltpu.VMEM_SHARED((NP, 16), jnp.float32),
        ],
    )(_sc_deg_body)
    return fn(dst2d, ew2d, z16)


def _sc_agg1_body(hp_hbm, src_hbm, dst_hbm, zeros_hbm, agg_hbm,
                  srcbuf, dstbuf, rows0, rows1, acc, sem0, sem1, ssem0, ssem1):
    c = lax.axis_index("c")
    s = lax.axis_index("s")
    wid = c * NS + s
    nb = s * RPS
    pltpu.sync_copy(zeros_hbm.at[pl.ds(nb, RPS)], acc.at[pl.ds(nb, RPS)])
    plsc.subcore_barrier()

    def stage(ob, carry):
        pltpu.sync_copy(src_hbm.at[wid, ob], srcbuf)
        pltpu.sync_copy(dst_hbm.at[wid, ob], dstbuf)
        pltpu.async_copy(hp_hbm.at[srcbuf.at[0]], rows0, sem0)
        pltpu.async_copy(hp_hbm.at[srcbuf.at[1]], rows1, sem1)

        def body(t, c2):
            j = t * 2
            pltpu.make_async_copy(hp_hbm.at[srcbuf.at[j]], rows0, sem0).wait()
            pltpu.async_copy(rows0, acc.at[dstbuf.at[j]], ssem0, add=True)
            pltpu.make_async_copy(hp_hbm.at[srcbuf.at[j + 1]], rows1, sem1).wait()
            pltpu.async_copy(rows1, acc.at[dstbuf.at[j + 1]], ssem1, add=True)
            pltpu.make_async_copy(rows0, acc.at[dstbuf.at[j]], ssem0).wait()
            pltpu.async_copy(hp_hbm.at[srcbuf.at[j + 2]], rows0, sem0)
            pltpu.make_async_copy(rows1, acc.at[dstbuf.at[j + 1]], ssem1).wait()
            pltpu.async_copy(hp_hbm.at[srcbuf.at[j + 3]], rows1, sem1)
            return c2

        lax.fori_loop(0, (SB - 3) // 2, body, 0)
        j = SB - 3
        pltpu.make_async_copy(hp_hbm.at[srcbuf.at[j]], rows0, sem0).wait()
        pltpu.async_copy(rows0, acc.at[dstbuf.at[j]], ssem0, add=True)
        pltpu.make_async_copy(hp_hbm.at[srcbuf.at[j + 1]], rows1, sem1).wait()
        pltpu.async_copy(rows1, acc.at[dstbuf.at[j + 1]], ssem1, add=True)
        pltpu.make_async_copy(rows0, acc.at[dstbuf.at[j]], ssem0).wait()
        pltpu.async_copy(hp_hbm.at[srcbuf.at[j + 2]], rows0, sem0)
        pltpu.make_async_copy(rows1, acc.at[dstbuf.at[j + 1]], ssem1).wait()
        pltpu.make_async_copy(hp_hbm.at[srcbuf.at[j + 2]], rows0, sem0).wait()
        pltpu.sync_copy(rows0, acc.at[dstbuf.at[j + 2]], add=True)
        return carry

    lax.fori_loop(0, NSTAGE, stage, 0)
    plsc.subcore_barrier()
    pltpu.sync_copy(acc.at[pl.ds(nb, RPS)], agg_hbm.at[c, pl.ds(nb, RPS)])


def _sc_agg1(hp, src2d, dst2d, z128):
    fn = functools.partial(
        pl.kernel,
        out_type=jax.ShapeDtypeStruct((NC, NP, D_HID), jnp.float32),
        mesh=_mesh(),
        scratch_types=[
            pltpu.VMEM((SB, CHUNK), jnp.int32),
            pltpu.VMEM((SB, CHUNK), jnp.int32),
            pltpu.VMEM((CHUNK, D_HID), jnp.float32),
            pltpu.VMEM((CHUNK, D_HID), jnp.float32),
            pltpu.VMEM_SHARED((NP, D_HID), jnp.float32),
            pltpu.SemaphoreType.DMA,
            pltpu.SemaphoreType.DMA,
            pltpu.SemaphoreType.DMA,
            pltpu.SemaphoreType.DMA,
        ],
    )(_sc_agg1_body)
    return fn(hp, src2d, dst2d, z128)


def _sc_agg2_body(zp_hbm, src_hbm, dst_hbm, ew_hbm, zeros_hbm, agg_hbm,
                  srcbuf, dstbuf, ewbuf, rows0, rows1, acc, sem0, sem1,
                  ssem0, ssem1):
    c = lax.axis_index("c")
    s = lax.axis_index("s")
    wid = c * NS + s
    pltpu.sync_copy(src_hbm.at[wid], srcbuf)
    pltpu.sync_copy(dst_hbm.at[wid], dstbuf)
    pltpu.sync_copy(ew_hbm.at[wid], ewbuf)
    nb = s * RPS
    pltpu.sync_copy(zeros_hbm.at[pl.ds(nb, RPS)], acc.at[pl.ds(nb, RPS)])
    plsc.subcore_barrier()

    def scale(rows_v, j):
        for g in range(CHUNK // 16):
            ewv = ewbuf[j, pl.ds(g * 16, 16)]
            for k in range(16):
                r = g * 16 + k
                rows_v[r] = rows_v[r] * ewv[k]

    pltpu.async_copy(zp_hbm.at[srcbuf.at[0]], rows0, sem0)
    pltpu.async_copy(zp_hbm.at[srcbuf.at[1]], rows1, sem1)

    def body(t, carry):
        j = t * 2
        pltpu.make_async_copy(zp_hbm.at[srcbuf.at[j]], rows0, sem0).wait()
        scale(rows0, j)
        pltpu.async_copy(rows0, acc.at[dstbuf.at[j]], ssem0, add=True)
        pltpu.make_async_copy(zp_hbm.at[srcbuf.at[j + 1]], rows1, sem1).wait()
        scale(rows1, j + 1)
        pltpu.async_copy(rows1, acc.at[dstbuf.at[j + 1]], ssem1, add=True)
        pltpu.make_async_copy(rows0, acc.at[dstbuf.at[j]], ssem0).wait()
        pltpu.async_copy(zp_hbm.at[srcbuf.at[j + 2]], rows0, sem0)
        pltpu.make_async_copy(rows1, acc.at[dstbuf.at[j + 1]], ssem1).wait()
        pltpu.async_copy(zp_hbm.at[srcbuf.at[j + 3]], rows1, sem1)
        return carry

    lax.fori_loop(0, (NCHUNK - 3) // 2, body, 0)
    j = NCHUNK - 3
    pltpu.make_async_copy(zp_hbm.at[srcbuf.at[j]], rows0, sem0).wait()
    scale(rows0, j)
    pltpu.async_copy(rows0, acc.at[dstbuf.at[j]], ssem0, add=True)
    pltpu.make_async_copy(zp_hbm.at[srcbuf.at[j + 1]], rows1, sem1).wait()
    scale(rows1, j + 1)
    pltpu.async_copy(rows1, acc.at[dstbuf.at[j + 1]], ssem1, add=True)
    pltpu.make_async_copy(rows0, acc.at[dstbuf.at[j]], ssem0).wait()
    pltpu.async_copy(zp_hbm.at[srcbuf.at[j + 2]], rows0, sem0)
    pltpu.make_async_copy(rows1, acc.at[dstbuf.at[j + 1]], ssem1).wait()
    pltpu.make_async_copy(zp_hbm.at[srcbuf.at[j + 2]], rows0, sem0).wait()
    scale(rows0, j + 2)
    pltpu.sync_copy(rows0, acc.at[dstbuf.at[j + 2]], add=True)
    plsc.subcore_barrier()
    pltpu.sync_copy(acc.at[pl.ds(nb, RPS)], agg_hbm.at[c, pl.ds(nb, RPS)])


def _sc_agg2(zp, src2d, dst2d, ew2d, z16):
    fn = functools.partial(
        pl.kernel,
        out_type=jax.ShapeDtypeStruct((NC, NP, D_OUT), jnp.float32),
        mesh=_mesh(),
        compiler_params=pltpu.CompilerParams(use_tc_tiling_on_sc=False),
        scratch_types=[
            pltpu.VMEM((NCHUNK, CHUNK), jnp.int32),
            pltpu.VMEM((NCHUNK, CHUNK), jnp.int32),
            pltpu.VMEM((NCHUNK, CHUNK), jnp.float32),
            pltpu.VMEM((CHUNK, D_OUT), jnp.float32),
            pltpu.VMEM((CHUNK, D_OUT), jnp.float32),
            pltpu.VMEM_SHARED((NP, D_OUT), jnp.float32),
            pltpu.SemaphoreType.DMA,
            pltpu.SemaphoreType.DMA,
            pltpu.SemaphoreType.DMA,
            pltpu.SemaphoreType.DMA,
        ],
    )(_sc_agg2_body)
    return fn(zp, src2d, dst2d, ew2d, z16)


def _tc_h_body(x_ref, w_ref, out_ref):
    out_ref[...] = jnp.dot(x_ref[...], w_ref[...],
                           preferred_element_type=jnp.float32)


def _tc_h(x, w1):
    return pl.pallas_call(
        _tc_h_body,
        grid=((N + BR - 1) // BR,),
        in_specs=[
            pl.BlockSpec((BR, D_IN), lambda i: (i, 0)),
            pl.BlockSpec((D_IN, D_HID), lambda i: (0, 0)),
        ],
        out_specs=pl.BlockSpec((BR, D_HID), lambda i: (i, 0)),
        out_shape=jax.ShapeDtypeStruct((N, D_HID), jnp.float32),
    )(x, w1)


def _tc_hp_body(h_ref, degp_ref, out_ref):
    deg1 = degp_ref[0, :, 0:1] + degp_ref[1, :, 0:1] + 1.0
    out_ref[...] = h_ref[...] * lax.rsqrt(deg1)


def _tc_hp(h, degp):
    return pl.pallas_call(
        _tc_hp_body,
        grid=((N + BR - 1) // BR,),
        in_specs=[
            pl.BlockSpec((BR, D_HID), lambda i: (i, 0)),
            pl.BlockSpec((NC, BR, 16), lambda i: (0, i, 0)),
        ],
        out_specs=pl.BlockSpec((BR, D_HID), lambda i: (i, 0)),
        out_shape=jax.ShapeDtypeStruct((N, D_HID), jnp.float32),
    )(h, degp)


def _tc_mid_body(agg_ref, hp_ref, degp_ref, w2_ref, b1_ref, out_ref):
    d1 = lax.rsqrt(degp_ref[0, :, 0:1] + degp_ref[1, :, 0:1] + 1.0)
    d2 = lax.rsqrt(degp_ref[0, :, 1:2] + degp_ref[1, :, 1:2] + 1.0)
    tot = agg_ref[0] + agg_ref[1] + hp_ref[...]
    x2 = jnp.maximum(tot * d1 + b1_ref[...][None, :], 0.0)
    z = jnp.dot(x2, w2_ref[...], preferred_element_type=jnp.float32)
    out_ref[...] = z * d2


def _tc_mid(agg1, hp, degp, w2, b1):
    return pl.pallas_call(
        _tc_mid_body,
        grid=((N + BR - 1) // BR,),
        in_specs=[
            pl.BlockSpec((NC, BR, D_HID), lambda i: (0, i, 0)),
            pl.BlockSpec((BR, D_HID), lambda i: (i, 0)),
            pl.BlockSpec((NC, BR, 16), lambda i: (0, i, 0)),
            pl.BlockSpec((D_HID, D_OUT), lambda i: (0, 0)),
            pl.BlockSpec((D_HID,), lambda i: (0,)),
        ],
        out_specs=pl.BlockSpec((BR, D_OUT), lambda i: (i, 0)),
        out_shape=jax.ShapeDtypeStruct((N, D_OUT), jnp.float32),
    )(agg1, hp, degp, w2, b1)


def _tc_out_body(agg_ref, zp_ref, degp_ref, b2_ref, out_ref):
    d2 = lax.rsqrt(degp_ref[0, :, 1:2] + degp_ref[1, :, 1:2] + 1.0)
    y = (agg_ref[0] + agg_ref[1] + zp_ref[...]) * d2 + b2_ref[...][None, :]
    m = jnp.max(y, axis=1, keepdims=True)
    ex = jnp.exp(y - m)
    out_ref[...] = y - m - jnp.log(jnp.sum(ex, axis=1, keepdims=True))


def _tc_out(agg2, zp, degp, b2):
    return pl.pallas_call(
        _tc_out_body,
        grid=((N + BR - 1) // BR,),
        in_specs=[
            pl.BlockSpec((NC, BR, D_OUT), lambda i: (0, i, 0)),
            pl.BlockSpec((BR, D_OUT), lambda i: (i, 0)),
            pl.BlockSpec((NC, BR, 16), lambda i: (0, i, 0)),
            pl.BlockSpec((D_OUT,), lambda i: (0,)),
        ],
        out_specs=pl.BlockSpec((BR, D_OUT), lambda i: (i, 0)),
        out_shape=jax.ShapeDtypeStruct((N, D_OUT), jnp.float32),
    )(agg2, zp, degp, b2)


def kernel(features, edge_index, edge_weight, W1, b1, W2, b2):
    src2d = edge_index[0].reshape(NW, NCHUNK, CHUNK)
    dst2d = edge_index[1].reshape(NW, NCHUNK, CHUNK)
    src4d = edge_index[0].reshape(NW, NSTAGE, SB, CHUNK)
    dst4d = edge_index[1].reshape(NW, NSTAGE, SB, CHUNK)
    ew2d = edge_weight.reshape(NW, NCHUNK, CHUNK)
    z16 = jnp.zeros((NP, 16), jnp.float32)
    z128 = jnp.zeros((NP, D_HID), jnp.float32)

    h = _tc_h(features, W1)
    degp = _sc_deg(dst2d, ew2d, z16)
    hp = _tc_hp(h, degp)
    agg1 = _sc_agg1(hp, src4d, dst4d, z128)
    zp = _tc_mid(agg1, hp, degp, W2, b1)
    agg2 = _sc_agg2(zp, src2d, dst2d, ew2d, z16)
    return _tc_out(agg2, zp, degp, b2)


# speedup vs baseline: 1.0723x; 1.0723x over previous
import functools

import jax
import jax.numpy as jnp
from jax import lax
from jax.experimental import pallas as pl
from jax.experimental.pallas import tpu as pltpu
from jax.experimental.pallas import tpu_sc as plsc

N = 10000
E = 320000
D_IN = 128
D_HID = 128
D_OUT = 16

NC = 2
NS = 16
NW = NC * NS
CHUNK = 80
EPW = E // NW
NCHUNK = EPW // CHUNK
SB = 25
NSTAGE = NCHUNK // SB
NP = 10240
RPS = NP // NS


def _mesh():
    return plsc.VectorSubcoreMesh(core_axis_name="c", subcore_axis_name="s")


def _sc_deg_body(dst_hbm, ew_hbm, zeros_hbm, degp_hbm, dstbuf, ewbuf, msg, acc):
    c = lax.axis_index("c")
    s = lax.axis_index("s")
    wid = c * NS + s
    nb = s * RPS
    pltpu.sync_copy(zeros_hbm.at[pl.ds(nb, RPS)], acc.at[pl.ds(nb, RPS)])
    lane = lax.iota(jnp.int32, 16)
    base_row = jnp.where(lane == 0, 1.0, 0.0).astype(jnp.float32)
    e1 = jnp.where(lane == 1, 1.0, 0.0).astype(jnp.float32)
    plsc.subcore_barrier()

    def stage(ob, carry):
        pltpu.sync_copy(dst_hbm.at[wid, ob], dstbuf)
        pltpu.sync_copy(ew_hbm.at[wid, ob], ewbuf)

        def body(j, c2):
            for g in range(CHUNK // 16):
                ewv = ewbuf[j, pl.ds(g * 16, 16)]
                for k in range(16):
                    msg[g * 16 + k] = base_row + e1 * ewv[k]
            pltpu.sync_copy(msg, acc.at[dstbuf.at[j]], add=True)
            return c2

        lax.fori_loop(0, SB, body, 0)
        return carry

    lax.fori_loop(0, NSTAGE, stage, 0)
    plsc.subcore_barrier()
    pltpu.sync_copy(acc.at[pl.ds(nb, RPS)], degp_hbm.at[c, pl.ds(nb, RPS)])


def _sc_deg(dst4d, ew4d, z16):
    fn = functools.partial(
        pl.kernel,
        out_type=jax.ShapeDtypeStruct((NC, NP, 16), jnp.float32),
        mesh=_mesh(),
        compiler_params=pltpu.CompilerParams(use_tc_tiling_on_sc=False),
        scratch_types=[
            pltpu.VMEM((SB, CHUNK), jnp.int32),
            pltpu.VMEM((SB, CHUNK), jnp.float32),
            pltpu.VMEM((CHUNK, 16), jnp.float32),
            pltpu.VMEM_SHARED((NP, 16), jnp.float32),
        ],
    )(_sc_deg_body)
    return fn(dst4d, ew4d, z16)


def _sc_agg1_body(hp_hbm, src_hbm, dst_hbm, zeros_hbm, agg_hbm,
                  srcbuf, dstbuf, rows0, rows1, acc, sem0, sem1):
    c = lax.axis_index("c")
    s = lax.axis_index("s")
    wid = c * NS + s
    nb = s * RPS
    pltpu.sync_copy(zeros_hbm.at[pl.ds(nb, RPS)], acc.at[pl.ds(nb, RPS)])
    plsc.subcore_barrier()

    def stage(ob, carry):
        pltpu.sync_copy(src_hbm.at[wid, ob], srcbuf)
        pltpu.sync_copy(dst_hbm.at[wid, ob], dstbuf)
        pltpu.async_copy(hp_hbm.at[srcbuf.at[0]], rows0, sem0)

        def body(t, c2):
            j = t * 2
            pltpu.async_copy(hp_hbm.at[srcbuf.at[j + 1]], rows1, sem1)
            pltpu.make_async_copy(hp_hbm.at[srcbuf.at[j]], rows0, sem0).wait()
            pltpu.sync_copy(rows0, acc.at[dstbuf.at[j]], add=True)
            pltpu.async_copy(hp_hbm.at[srcbuf.at[j + 2]], rows0, sem0)
            pltpu.make_async_copy(hp_hbm.at[srcbuf.at[j + 1]], rows1, sem1).wait()
            pltpu.sync_copy(rows1, acc.at[dstbuf.at[j + 1]], add=True)
            return c2

        lax.fori_loop(0, (SB - 1) // 2, body, 0)
        pltpu.make_async_copy(hp_hbm.at[srcbuf.at[SB - 1]], rows0, sem0).wait()
        pltpu.sync_copy(rows0, acc.at[dstbuf.at[SB - 1]], add=True)
        return carry

    lax.fori_loop(0, NSTAGE, stage, 0)
    plsc.subcore_barrier()
    pltpu.sync_copy(acc.at[pl.ds(nb, RPS)], agg_hbm.at[c, pl.ds(nb, RPS)])


def _sc_agg1(hp, src4d, dst4d, z128):
    fn = functools.partial(
        pl.kernel,
        out_type=jax.ShapeDtypeStruct((NC, NP, D_HID), jnp.float32),
        mesh=_mesh(),
        scratch_types=[
            pltpu.VMEM((SB, CHUNK), jnp.int32),
            pltpu.VMEM((SB, CHUNK), jnp.int32),
            pltpu.VMEM((CHUNK, D_HID), jnp.float32),
            pltpu.VMEM((CHUNK, D_HID), jnp.float32),
            pltpu.VMEM_SHARED((NP, D_HID), jnp.float32),
            pltpu.SemaphoreType.DMA,
            pltpu.SemaphoreType.DMA,
        ],
    )(_sc_agg1_body)
    return fn(hp, src4d, dst4d, z128)


def _sc_agg2_body(zp_hbm, src_hbm, dst_hbm, ew_hbm, zeros_hbm, agg_hbm,
                  srcbuf, dstbuf, ewbuf, rows0, rows1, acc, sem0, sem1):
    c = lax.axis_index("c")
    s = lax.axis_index("s")
    wid = c * NS + s
    nb = s * RPS
    pltpu.sync_copy(zeros_hbm.at[pl.ds(nb, RPS)], acc.at[pl.ds(nb, RPS)])
    plsc.subcore_barrier()

    def scale(rows_v, j):
        for g in range(CHUNK // 16):
            ewv = ewbuf[j, pl.ds(g * 16, 16)]
            for k in range(16):
                r = g * 16 + k
                rows_v[r] = rows_v[r] * ewv[k]

    def stage(ob, carry):
        pltpu.sync_copy(src_hbm.at[wid, ob], srcbuf)
        pltpu.sync_copy(dst_hbm.at[wid, ob], dstbuf)
        pltpu.sync_copy(ew_hbm.at[wid, ob], ewbuf)
        pltpu.async_copy(zp_hbm.at[srcbuf.at[0]], rows0, sem0)

        def body(t, c2):
            j = t * 2
            pltpu.async_copy(zp_hbm.at[srcbuf.at[j + 1]], rows1, sem1)
            pltpu.make_async_copy(zp_hbm.at[srcbuf.at[j]], rows0, sem0).wait()
            scale(rows0, j)
            pltpu.sync_copy(rows0, acc.at[dstbuf.at[j]], add=True)
            pltpu.async_copy(zp_hbm.at[srcbuf.at[j + 2]], rows0, sem0)
            pltpu.make_async_copy(zp_hbm.at[srcbuf.at[j + 1]], rows1, sem1).wait()
            scale(rows1, j + 1)
            pltpu.sync_copy(rows1, acc.at[dstbuf.at[j + 1]], add=True)
            return c2

        lax.fori_loop(0, (SB - 1) // 2, body, 0)
        pltpu.make_async_copy(zp_hbm.at[srcbuf.at[SB - 1]], rows0, sem0).wait()
        scale(rows0, SB - 1)
        pltpu.sync_copy(rows0, acc.at[dstbuf.at[SB - 1]], add=True)
        return carry

    lax.fori_loop(0, NSTAGE, stage, 0)
    plsc.subcore_barrier()
    pltpu.sync_copy(acc.at[pl.ds(nb, RPS)], agg_hbm.at[c, pl.ds(nb, RPS)])


def _sc_agg2(zp, src4d, dst4d, ew4d, z16):
    fn = functools.partial(
        pl.kernel,
        out_type=jax.ShapeDtypeStruct((NC, NP, D_OUT), jnp.float32),
        mesh=_mesh(),
        compiler_params=pltpu.CompilerParams(use_tc_tiling_on_sc=False),
        scratch_types=[
            pltpu.VMEM((SB, CHUNK), jnp.int32),
            pltpu.VMEM((SB, CHUNK), jnp.int32),
            pltpu.VMEM((SB, CHUNK), jnp.float32),
            pltpu.VMEM((CHUNK, D_OUT), jnp.float32),
            pltpu.VMEM((CHUNK, D_OUT), jnp.float32),
            pltpu.VMEM_SHARED((NP, D_OUT), jnp.float32),
            pltpu.SemaphoreType.DMA,
            pltpu.SemaphoreType.DMA,
        ],
    )(_sc_agg2_body)
    return fn(zp, src4d, dst4d, ew4d, z16)


def _tc_h_body(x_ref, w_ref, out_ref):
    out_ref[...] = jnp.dot(x_ref[...], w_ref[...],
                           preferred_element_type=jnp.float32)


def _tc_h(x, w1):
    return pl.pallas_call(
        _tc_h_body,
        out_shape=jax.ShapeDtypeStruct((N, D_HID), jnp.float32),
    )(x, w1)


def _tc_hp_body(h_ref, degp_ref, out_ref):
    deg1 = degp_ref[0, 0:N, 0:1] + degp_ref[1, 0:N, 0:1] + 1.0
    out_ref[...] = h_ref[...] * lax.rsqrt(deg1)


def _tc_hp(h, degp):
    return pl.pallas_call(
        _tc_hp_body,
        out_shape=jax.ShapeDtypeStruct((N, D_HID), jnp.float32),
    )(h, degp)


def _tc_mid_body(agg_ref, hp_ref, degp_ref, w2_ref, b1_ref, out_ref):
    d1 = lax.rsqrt(degp_ref[0, 0:N, 0:1] + degp_ref[1, 0:N, 0:1] + 1.0)
    d2 = lax.rsqrt(degp_ref[0, 0:N, 1:2] + degp_ref[1, 0:N, 1:2] + 1.0)
    tot = agg_ref[0, 0:N, :] + agg_ref[1, 0:N, :] + hp_ref[...]
    x2 = jnp.maximum(tot * d1 + b1_ref[...][None, :], 0.0)
    z = jnp.dot(x2, w2_ref[...], preferred_element_type=jnp.float32)
    out_ref[...] = z * d2


def _tc_mid(agg1, hp, degp, w2, b1):
    return pl.pallas_call(
        _tc_mid_body,
        out_shape=jax.ShapeDtypeStruct((N, D_OUT), jnp.float32),
    )(agg1, hp, degp, w2, b1)


def _tc_out_body(agg_ref, zp_ref, degp_ref, b2_ref, out_ref):
    d2 = lax.rsqrt(degp_ref[0, 0:N, 1:2] + degp_ref[1, 0:N, 1:2] + 1.0)
    y = (agg_ref[0, 0:N, :] + agg_ref[1, 0:N, :] + zp_ref[...]) * d2
    y = y + b2_ref[...][None, :]
    m = jnp.max(y, axis=1, keepdims=True)
    ex = jnp.exp(y - m)
    out_ref[...] = y - m - jnp.log(jnp.sum(ex, axis=1, keepdims=True))


def _tc_out(agg2, zp, degp, b2):
    return pl.pallas_call(
        _tc_out_body,
        out_shape=jax.ShapeDtypeStruct((N, D_OUT), jnp.float32),
    )(agg2, zp, degp, b2)


def kernel(features, edge_index, edge_weight, W1, b1, W2, b2):
    src4d = edge_index[0].reshape(NW, NSTAGE, SB, CHUNK)
    dst4d = edge_index[1].reshape(NW, NSTAGE, SB, CHUNK)
    ew4d = edge_weight.reshape(NW, NSTAGE, SB, CHUNK)
    z16 = jnp.zeros((NP, 16), jnp.float32)
    z128 = jnp.zeros((NP, D_HID), jnp.float32)

    h = _tc_h(features, W1)
    degp = _sc_deg(dst4d, ew4d, z16)
    hp = _tc_hp(h, degp)
    agg1 = _sc_agg1(hp, src4d, dst4d, z128)
    zp = _tc_mid(agg1, hp, degp, W2, b1)
    agg2 = _sc_agg2(zp, src4d, dst4d, ew4d, z16)
    return _tc_out(agg2, zp, degp, b2)


# speedup vs baseline: 1.1042x; 1.0298x over previous
import functools

import jax
import jax.numpy as jnp
from jax import lax
from jax.experimental import pallas as pl
from jax.experimental.pallas import tpu as pltpu
from jax.experimental.pallas import tpu_sc as plsc

N = 10000
E = 320000
D_IN = 128
D_HID = 128
D_OUT = 16

NC = 2
NS = 16
NW = NC * NS
CHUNK = 80
EPW = E // NW
NCHUNK = EPW // CHUNK
SB = 25
NSTAGE = NCHUNK // SB
NP = 10240
RPS = NP // NS


def _mesh():
    return plsc.VectorSubcoreMesh(core_axis_name="c", subcore_axis_name="s")


def _sc_deg_body(dst_hbm, ew_hbm, zeros_hbm, degp_hbm, dstbuf, ewbuf, msg, acc):
    c = lax.axis_index("c")
    s = lax.axis_index("s")
    wid = c * NS + s
    nb = s * RPS
    pltpu.sync_copy(zeros_hbm.at[pl.ds(nb, RPS)], acc.at[pl.ds(nb, RPS)])
    lane = lax.iota(jnp.int32, 16)
    base_row = jnp.where(lane == 0, 1.0, 0.0).astype(jnp.float32)
    e1 = jnp.where(lane == 1, 1.0, 0.0).astype(jnp.float32)
    plsc.subcore_barrier()

    pltpu.sync_copy(dst_hbm.at[wid], dstbuf)
    pltpu.sync_copy(ew_hbm.at[wid], ewbuf)

    def stage(ob, carry):
        def body(j, c2):
            for g in range(CHUNK // 16):
                ewv = ewbuf[ob, j, pl.ds(g * 16, 16)]
                for k in range(16):
                    msg[g * 16 + k] = base_row + e1 * ewv[k]
            pltpu.sync_copy(msg, acc.at[dstbuf.at[ob, j]], add=True)
            return c2

        lax.fori_loop(0, SB, body, 0)
        return carry

    lax.fori_loop(0, NSTAGE, stage, 0)
    plsc.subcore_barrier()
    pltpu.sync_copy(acc.at[pl.ds(nb, RPS)], degp_hbm.at[c, pl.ds(nb, RPS)])


def _sc_deg(dst4d, ew4d, z16):
    fn = functools.partial(
        pl.kernel,
        out_type=jax.ShapeDtypeStruct((NC, NP, 16), jnp.float32),
        mesh=_mesh(),
        compiler_params=pltpu.CompilerParams(use_tc_tiling_on_sc=False),
        scratch_types=[
            pltpu.VMEM((NSTAGE, SB, CHUNK), jnp.int32),
            pltpu.VMEM((NSTAGE, SB, CHUNK), jnp.float32),
            pltpu.VMEM((CHUNK, 16), jnp.float32),
            pltpu.VMEM_SHARED((NP, 16), jnp.float32),
        ],
    )(_sc_deg_body)
    return fn(dst4d, ew4d, z16)


def _sc_agg1_body(hp_hbm, src_hbm, dst_hbm, zeros_hbm, agg_hbm,
                  srcbuf, dstbuf, rows0, rows1, acc, sem0, sem1):
    c = lax.axis_index("c")
    s = lax.axis_index("s")
    wid = c * NS + s
    nb = s * RPS
    pltpu.sync_copy(zeros_hbm.at[pl.ds(nb, RPS)], acc.at[pl.ds(nb, RPS)])
    plsc.subcore_barrier()

    def stage(ob, carry):
        pltpu.sync_copy(src_hbm.at[wid, ob], srcbuf)
        pltpu.sync_copy(dst_hbm.at[wid, ob], dstbuf)
        pltpu.async_copy(hp_hbm.at[srcbuf.at[0]], rows0, sem0)

        def body(t, c2):
            j = t * 2
            pltpu.async_copy(hp_hbm.at[srcbuf.at[j + 1]], rows1, sem1)
            pltpu.make_async_copy(hp_hbm.at[srcbuf.at[j]], rows0, sem0).wait()
            pltpu.sync_copy(rows0, acc.at[dstbuf.at[j]], add=True)
            pltpu.async_copy(hp_hbm.at[srcbuf.at[j + 2]], rows0, sem0)
            pltpu.make_async_copy(hp_hbm.at[srcbuf.at[j + 1]], rows1, sem1).wait()
            pltpu.sync_copy(rows1, acc.at[dstbuf.at[j + 1]], add=True)
            return c2

        lax.fori_loop(0, (SB - 1) // 2, body, 0)
        pltpu.make_async_copy(hp_hbm.at[srcbuf.at[SB - 1]], rows0, sem0).wait()
        pltpu.sync_copy(rows0, acc.at[dstbuf.at[SB - 1]], add=True)
        return carry

    lax.fori_loop(0, NSTAGE, stage, 0)
    plsc.subcore_barrier()
    pltpu.sync_copy(acc.at[pl.ds(nb, RPS)], agg_hbm.at[c, pl.ds(nb, RPS)])


def _sc_agg1(hp, src4d, dst4d, z128):
    fn = functools.partial(
        pl.kernel,
        out_type=jax.ShapeDtypeStruct((NC, NP, D_HID), jnp.float32),
        mesh=_mesh(),
        scratch_types=[
            pltpu.VMEM((SB, CHUNK), jnp.int32),
            pltpu.VMEM((SB, CHUNK), jnp.int32),
            pltpu.VMEM((CHUNK, D_HID), jnp.float32),
            pltpu.VMEM((CHUNK, D_HID), jnp.float32),
            pltpu.VMEM_SHARED((NP, D_HID), jnp.float32),
            pltpu.SemaphoreType.DMA,
            pltpu.SemaphoreType.DMA,
        ],
    )(_sc_agg1_body)
    return fn(hp, src4d, dst4d, z128)


def _sc_agg2_body(zp_hbm, src_hbm, dst_hbm, ew_hbm, zeros_hbm, agg_hbm,
                  srcbuf, dstbuf, ewbuf, rows0, rows1, acc, sem0, sem1):
    c = lax.axis_index("c")
    s = lax.axis_index("s")
    wid = c * NS + s
    nb = s * RPS
    pltpu.sync_copy(zeros_hbm.at[pl.ds(nb, RPS)], acc.at[pl.ds(nb, RPS)])
    plsc.subcore_barrier()

    pltpu.sync_copy(src_hbm.at[wid], srcbuf)
    pltpu.sync_copy(dst_hbm.at[wid], dstbuf)
    pltpu.sync_copy(ew_hbm.at[wid], ewbuf)

    def scale(rows_v, ob, j):
        for g in range(CHUNK // 16):
            ewv = ewbuf[ob, j, pl.ds(g * 16, 16)]
            for k in range(16):
                r = g * 16 + k
                rows_v[r] = rows_v[r] * ewv[k]

    def stage(ob, carry):
        pltpu.async_copy(zp_hbm.at[srcbuf.at[ob, 0]], rows0, sem0)

        def body(t, c2):
            j = t * 2
            pltpu.async_copy(zp_hbm.at[srcbuf.at[ob, j + 1]], rows1, sem1)
            pltpu.make_async_copy(zp_hbm.at[srcbuf.at[ob, j]], rows0, sem0).wait()
            scale(rows0, ob, j)
            pltpu.sync_copy(rows0, acc.at[dstbuf.at[ob, j]], add=True)
            pltpu.async_copy(zp_hbm.at[srcbuf.at[ob, j + 2]], rows0, sem0)
            pltpu.make_async_copy(zp_hbm.at[srcbuf.at[ob, j + 1]], rows1, sem1).wait()
            scale(rows1, ob, j + 1)
            pltpu.sync_copy(rows1, acc.at[dstbuf.at[ob, j + 1]], add=True)
            return c2

        lax.fori_loop(0, (SB - 1) // 2, body, 0)
        pltpu.make_async_copy(zp_hbm.at[srcbuf.at[ob, SB - 1]], rows0, sem0).wait()
        scale(rows0, ob, SB - 1)
        pltpu.sync_copy(rows0, acc.at[dstbuf.at[ob, SB - 1]], add=True)
        return carry

    lax.fori_loop(0, NSTAGE, stage, 0)
    plsc.subcore_barrier()
    pltpu.sync_copy(acc.at[pl.ds(nb, RPS)], agg_hbm.at[c, pl.ds(nb, RPS)])


def _sc_agg2(zp, src4d, dst4d, ew4d, z16):
    fn = functools.partial(
        pl.kernel,
        out_type=jax.ShapeDtypeStruct((NC, NP, D_OUT), jnp.float32),
        mesh=_mesh(),
        compiler_params=pltpu.CompilerParams(use_tc_tiling_on_sc=False),
        scratch_types=[
            pltpu.VMEM((NSTAGE, SB, CHUNK), jnp.int32),
            pltpu.VMEM((NSTAGE, SB, CHUNK), jnp.int32),
            pltpu.VMEM((NSTAGE, SB, CHUNK), jnp.float32),
            pltpu.VMEM((CHUNK, D_OUT), jnp.float32),
            pltpu.VMEM((CHUNK, D_OUT), jnp.float32),
            pltpu.VMEM_SHARED((NP, D_OUT), jnp.float32),
            pltpu.SemaphoreType.DMA,
            pltpu.SemaphoreType.DMA,
        ],
    )(_sc_agg2_body)
    return fn(zp, src4d, dst4d, ew4d, z16)


def _tc_h_body(x_ref, w_ref, out_ref):
    out_ref[...] = jnp.dot(x_ref[...], w_ref[...],
                           preferred_element_type=jnp.float32)


def _tc_h(x, w1):
    return pl.pallas_call(
        _tc_h_body,
        out_shape=jax.ShapeDtypeStruct((N, D_HID), jnp.float32),
    )(x, w1)


def _tc_hp_body(h_ref, degp_ref, out_ref):
    deg1 = degp_ref[0, 0:N, 0:1] + degp_ref[1, 0:N, 0:1] + 1.0
    out_ref[...] = h_ref[...] * lax.rsqrt(deg1)


def _tc_hp(h, degp):
    return pl.pallas_call(
        _tc_hp_body,
        out_shape=jax.ShapeDtypeStruct((N, D_HID), jnp.float32),
    )(h, degp)


def _tc_mid_body(agg_ref, hp_ref, degp_ref, w2_ref, b1_ref, out_ref):
    d1 = lax.rsqrt(degp_ref[0, 0:N, 0:1] + degp_ref[1, 0:N, 0:1] + 1.0)
    d2 = lax.rsqrt(degp_ref[0, 0:N, 1:2] + degp_ref[1, 0:N, 1:2] + 1.0)
    tot = agg_ref[0, 0:N, :] + agg_ref[1, 0:N, :] + hp_ref[...]
    x2 = jnp.maximum(tot * d1 + b1_ref[...][None, :], 0.0)
    z = jnp.dot(x2, w2_ref[...], preferred_element_type=jnp.float32)
    out_ref[...] = z * d2


def _tc_mid(agg1, hp, degp, w2, b1):
    return pl.pallas_call(
        _tc_mid_body,
        out_shape=jax.ShapeDtypeStruct((N, D_OUT), jnp.float32),
    )(agg1, hp, degp, w2, b1)


def _tc_out_body(agg_ref, zp_ref, degp_ref, b2_ref, out_ref):
    d2 = lax.rsqrt(degp_ref[0, 0:N, 1:2] + degp_ref[1, 0:N, 1:2] + 1.0)
    y = (agg_ref[0, 0:N, :] + agg_ref[1, 0:N, :] + zp_ref[...]) * d2
    y = y + b2_ref[...][None, :]
    m = jnp.max(y, axis=1, keepdims=True)
    ex = jnp.exp(y - m)
    out_ref[...] = y - m - jnp.log(jnp.sum(ex, axis=1, keepdims=True))


def _tc_out(agg2, zp, degp, b2):
    return pl.pallas_call(
        _tc_out_body,
        out_shape=jax.ShapeDtypeStruct((N, D_OUT), jnp.float32),
    )(agg2, zp, degp, b2)


def kernel(features, edge_index, edge_weight, W1, b1, W2, b2):
    src4d = edge_index[0].reshape(NW, NSTAGE, SB, CHUNK)
    dst4d = edge_index[1].reshape(NW, NSTAGE, SB, CHUNK)
    ew4d = edge_weight.reshape(NW, NSTAGE, SB, CHUNK)
    z16 = jnp.zeros((NP, 16), jnp.float32)
    z128 = jnp.zeros((NP, D_HID), jnp.float32)

    h = _tc_h(features, W1)
    degp = _sc_deg(dst4d, ew4d, z16)
    hp = _tc_hp(h, degp)
    agg1 = _sc_agg1(hp, src4d, dst4d, z128)
    zp = _tc_mid(agg1, hp, degp, W2, b1)
    agg2 = _sc_agg2(zp, src4d, dst4d, ew4d, z16)
    return _tc_out(agg2, zp, degp, b2)


# speedup vs baseline: 1.1070x; 1.0025x over previous
import functools

import jax
import jax.numpy as jnp
from jax import lax
from jax.experimental import pallas as pl
from jax.experimental.pallas import tpu as pltpu
from jax.experimental.pallas import tpu_sc as plsc

N = 10000
E = 320000
D_IN = 128
D_HID = 128
D_OUT = 16

NC = 2
NS = 16
NW = NC * NS
CHUNK = 80
EPW = E // NW
NCHUNK = EPW // CHUNK
SB = 25
NSTAGE = NCHUNK // SB
NP = 10240
RPS = NP // NS


def _mesh():
    return plsc.VectorSubcoreMesh(core_axis_name="c", subcore_axis_name="s")


def _sc_deg_body(dst_hbm, ew_hbm, zeros_hbm, degp_hbm, dstbuf, ewbuf, msg, acc):
    c = lax.axis_index("c")
    s = lax.axis_index("s")
    wid = c * NS + s
    nb = s * RPS
    pltpu.sync_copy(zeros_hbm.at[pl.ds(nb, RPS)], acc.at[pl.ds(nb, RPS)])
    lane = lax.iota(jnp.int32, 16)
    base_row = jnp.where(lane == 0, 1.0, 0.0).astype(jnp.float32)
    e1 = jnp.where(lane == 1, 1.0, 0.0).astype(jnp.float32)
    plsc.subcore_barrier()

    pltpu.sync_copy(dst_hbm.at[wid], dstbuf)
    pltpu.sync_copy(ew_hbm.at[wid], ewbuf)

    def stage(ob, carry):
        def body(j, c2):
            for g in range(CHUNK // 16):
                ewv = ewbuf[ob, j, pl.ds(g * 16, 16)]
                for k in range(16):
                    msg[g * 16 + k] = base_row + e1 * ewv[k]
            pltpu.sync_copy(msg, acc.at[dstbuf.at[ob, j]], add=True)
            return c2

        lax.fori_loop(0, SB, body, 0)
        return carry

    lax.fori_loop(0, NSTAGE, stage, 0)
    plsc.subcore_barrier()
    pltpu.sync_copy(acc.at[pl.ds(nb, RPS)], degp_hbm.at[c, pl.ds(nb, RPS)])


def _sc_deg(dst4d, ew4d, z16):
    fn = functools.partial(
        pl.kernel,
        out_type=jax.ShapeDtypeStruct((NC, NP, 16), jnp.float32),
        mesh=_mesh(),
        compiler_params=pltpu.CompilerParams(use_tc_tiling_on_sc=False),
        scratch_types=[
            pltpu.VMEM((NSTAGE, SB, CHUNK), jnp.int32),
            pltpu.VMEM((NSTAGE, SB, CHUNK), jnp.float32),
            pltpu.VMEM((CHUNK, 16), jnp.float32),
            pltpu.VMEM_SHARED((NP, 16), jnp.float32),
        ],
    )(_sc_deg_body)
    return fn(dst4d, ew4d, z16)


def _sc_agg1_body(hp_hbm, src_hbm, dst_hbm, zeros_hbm, agg_hbm,
                  srcbuf, dstbuf, rows0, rows1, acc, sem0, sem1):
    c = lax.axis_index("c")
    s = lax.axis_index("s")
    wid = c * NS + s
    nb = s * RPS
    pltpu.sync_copy(zeros_hbm.at[pl.ds(nb, RPS)], acc.at[pl.ds(nb, RPS)])
    plsc.subcore_barrier()

    def stage(ob, carry):
        pltpu.sync_copy(src_hbm.at[wid, ob], srcbuf)
        pltpu.sync_copy(dst_hbm.at[wid, ob], dstbuf)
        pltpu.async_copy(hp_hbm.at[srcbuf.at[0]], rows0, sem0)

        def body(t, c2):
            j = t * 2
            pltpu.async_copy(hp_hbm.at[srcbuf.at[j + 1]], rows1, sem1)
            pltpu.make_async_copy(hp_hbm.at[srcbuf.at[j]], rows0, sem0).wait()
            pltpu.sync_copy(rows0, acc.at[dstbuf.at[j]], add=True)
            pltpu.async_copy(hp_hbm.at[srcbuf.at[j + 2]], rows0, sem0)
            pltpu.make_async_copy(hp_hbm.at[srcbuf.at[j + 1]], rows1, sem1).wait()
            pltpu.sync_copy(rows1, acc.at[dstbuf.at[j + 1]], add=True)
            return c2

        lax.fori_loop(0, (SB - 1) // 2, body, 0)
        pltpu.make_async_copy(hp_hbm.at[srcbuf.at[SB - 1]], rows0, sem0).wait()
        pltpu.sync_copy(rows0, acc.at[dstbuf.at[SB - 1]], add=True)
        return carry

    lax.fori_loop(0, NSTAGE, stage, 0)
    plsc.subcore_barrier()
    pltpu.sync_copy(acc.at[pl.ds(nb, RPS)], agg_hbm.at[c, pl.ds(nb, RPS)])


def _sc_agg1(hp, src4d, dst4d, z128):
    fn = functools.partial(
        pl.kernel,
        out_type=jax.ShapeDtypeStruct((NC, NP, D_HID), jnp.float32),
        mesh=_mesh(),
        compiler_params=pltpu.CompilerParams(use_tc_tiling_on_sc=False),
        scratch_types=[
            pltpu.VMEM((SB, CHUNK), jnp.int32),
            pltpu.VMEM((SB, CHUNK), jnp.int32),
            pltpu.VMEM((CHUNK, D_HID), jnp.float32),
            pltpu.VMEM((CHUNK, D_HID), jnp.float32),
            pltpu.VMEM_SHARED((NP, D_HID), jnp.float32),
            pltpu.SemaphoreType.DMA,
            pltpu.SemaphoreType.DMA,
        ],
    )(_sc_agg1_body)
    return fn(hp, src4d, dst4d, z128)


def _sc_agg2_body(zp_hbm, src_hbm, dst_hbm, ew_hbm, zeros_hbm, agg_hbm,
                  srcbuf, dstbuf, ewbuf, rows0, rows1, acc, sem0, sem1):
    c = lax.axis_index("c")
    s = lax.axis_index("s")
    wid = c * NS + s
    nb = s * RPS
    pltpu.sync_copy(zeros_hbm.at[pl.ds(nb, RPS)], acc.at[pl.ds(nb, RPS)])
    plsc.subcore_barrier()

    pltpu.sync_copy(src_hbm.at[wid], srcbuf)
    pltpu.sync_copy(dst_hbm.at[wid], dstbuf)
    pltpu.sync_copy(ew_hbm.at[wid], ewbuf)

    def scale(rows_v, ob, j):
        for g in range(CHUNK // 16):
            ewv = ewbuf[ob, j, pl.ds(g * 16, 16)]
            for k in range(16):
                r = g * 16 + k
                rows_v[r] = rows_v[r] * ewv[k]

    def stage(ob, carry):
        pltpu.async_copy(zp_hbm.at[srcbuf.at[ob, 0]], rows0, sem0)

        def body(t, c2):
            j = t * 2
            pltpu.async_copy(zp_hbm.at[srcbuf.at[ob, j + 1]], rows1, sem1)
            pltpu.make_async_copy(zp_hbm.at[srcbuf.at[ob, j]], rows0, sem0).wait()
            scale(rows0, ob, j)
            pltpu.sync_copy(rows0, acc.at[dstbuf.at[ob, j]], add=True)
            pltpu.async_copy(zp_hbm.at[srcbuf.at[ob, j + 2]], rows0, sem0)
            pltpu.make_async_copy(zp_hbm.at[srcbuf.at[ob, j + 1]], rows1, sem1).wait()
            scale(rows1, ob, j + 1)
            pltpu.sync_copy(rows1, acc.at[dstbuf.at[ob, j + 1]], add=True)
            return c2

        lax.fori_loop(0, (SB - 1) // 2, body, 0)
        pltpu.make_async_copy(zp_hbm.at[srcbuf.at[ob, SB - 1]], rows0, sem0).wait()
        scale(rows0, ob, SB - 1)
        pltpu.sync_copy(rows0, acc.at[dstbuf.at[ob, SB - 1]], add=True)
        return carry

    lax.fori_loop(0, NSTAGE, stage, 0)
    plsc.subcore_barrier()
    pltpu.sync_copy(acc.at[pl.ds(nb, RPS)], agg_hbm.at[c, pl.ds(nb, RPS)])


def _sc_agg2(zp, src4d, dst4d, ew4d, z16):
    fn = functools.partial(
        pl.kernel,
        out_type=jax.ShapeDtypeStruct((NC, NP, D_OUT), jnp.float32),
        mesh=_mesh(),
        compiler_params=pltpu.CompilerParams(use_tc_tiling_on_sc=False),
        scratch_types=[
            pltpu.VMEM((NSTAGE, SB, CHUNK), jnp.int32),
            pltpu.VMEM((NSTAGE, SB, CHUNK), jnp.int32),
            pltpu.VMEM((NSTAGE, SB, CHUNK), jnp.float32),
            pltpu.VMEM((CHUNK, D_OUT), jnp.float32),
            pltpu.VMEM((CHUNK, D_OUT), jnp.float32),
            pltpu.VMEM_SHARED((NP, D_OUT), jnp.float32),
            pltpu.SemaphoreType.DMA,
            pltpu.SemaphoreType.DMA,
        ],
    )(_sc_agg2_body)
    return fn(zp, src4d, dst4d, ew4d, z16)


def _tc_h_body(x_ref, w_ref, out_ref):
    out_ref[...] = jnp.dot(x_ref[...], w_ref[...],
                           preferred_element_type=jnp.float32)


def _tc_h(x, w1):
    return pl.pallas_call(
        _tc_h_body,
        out_shape=jax.ShapeDtypeStruct((N, D_HID), jnp.float32),
    )(x, w1)


def _tc_hp_body(h_ref, degp_ref, out_ref):
    deg1 = degp_ref[0, 0:N, 0:1] + degp_ref[1, 0:N, 0:1] + 1.0
    out_ref[...] = h_ref[...] * lax.rsqrt(deg1)


def _tc_hp(h, degp):
    return pl.pallas_call(
        _tc_hp_body,
        out_shape=jax.ShapeDtypeStruct((N, D_HID), jnp.float32),
    )(h, degp)


def _tc_mid_body(agg_ref, hp_ref, degp_ref, w2_ref, b1_ref, out_ref):
    d1 = lax.rsqrt(degp_ref[0, 0:N, 0:1] + degp_ref[1, 0:N, 0:1] + 1.0)
    d2 = lax.rsqrt(degp_ref[0, 0:N, 1:2] + degp_ref[1, 0:N, 1:2] + 1.0)
    tot = agg_ref[0, 0:N, :] + agg_ref[1, 0:N, :] + hp_ref[...]
    x2 = jnp.maximum(tot * d1 + b1_ref[...][None, :], 0.0)
    z = jnp.dot(x2, w2_ref[...], preferred_element_type=jnp.float32)
    out_ref[...] = z * d2


def _tc_mid(agg1, hp, degp, w2, b1):
    return pl.pallas_call(
        _tc_mid_body,
        out_shape=jax.ShapeDtypeStruct((N, D_OUT), jnp.float32),
    )(agg1, hp, degp, w2, b1)


def _tc_out_body(agg_ref, zp_ref, degp_ref, b2_ref, out_ref):
    d2 = lax.rsqrt(degp_ref[0, 0:N, 1:2] + degp_ref[1, 0:N, 1:2] + 1.0)
    y = (agg_ref[0, 0:N, :] + agg_ref[1, 0:N, :] + zp_ref[...]) * d2
    y = y + b2_ref[...][None, :]
    m = jnp.max(y, axis=1, keepdims=True)
    ex = jnp.exp(y - m)
    out_ref[...] = y - m - jnp.log(jnp.sum(ex, axis=1, keepdims=True))


def _tc_out(agg2, zp, degp, b2):
    return pl.pallas_call(
        _tc_out_body,
        out_shape=jax.ShapeDtypeStruct((N, D_OUT), jnp.float32),
    )(agg2, zp, degp, b2)


def kernel(features, edge_index, edge_weight, W1, b1, W2, b2):
    src4d = edge_index[0].reshape(NW, NSTAGE, SB, CHUNK)
    dst4d = edge_index[1].reshape(NW, NSTAGE, SB, CHUNK)
    ew4d = edge_weight.reshape(NW, NSTAGE, SB, CHUNK)
    z16 = jnp.zeros((NP, 16), jnp.float32)
    z128 = jnp.zeros((NP, D_HID), jnp.float32)

    h = _tc_h(features, W1)
    degp = _sc_deg(dst4d, ew4d, z16)
    hp = _tc_hp(h, degp)
    agg1 = _sc_agg1(hp, src4d, dst4d, z128)
    zp = _tc_mid(agg1, hp, degp, W2, b1)
    agg2 = _sc_agg2(zp, src4d, dst4d, ew4d, z16)
    return _tc_out(agg2, zp, degp, b2)


# speedup vs baseline: 1.1437x; 1.0332x over previous
import functools

import jax
import jax.numpy as jnp
from jax import lax
from jax.experimental import pallas as pl
from jax.experimental.pallas import tpu as pltpu
from jax.experimental.pallas import tpu_sc as plsc

N = 10000
E = 320000
D_IN = 128
D_HID = 128
D_OUT = 16

NC = 2
NS = 16
NW = NC * NS
CHUNK = 80
EPW = E // NW
NCHUNK = EPW // CHUNK
SB = 25
NSTAGE = NCHUNK // SB
NP = 10240
RPS = NP // NS


def _mesh():
    return plsc.VectorSubcoreMesh(core_axis_name="c", subcore_axis_name="s")


def _sc_deg_body(ei_hbm, ew_hbm, zeros_hbm, degp_hbm, dstbuf, ewbuf, msg, acc):
    c = lax.axis_index("c")
    s = lax.axis_index("s")
    wid = c * NS + s
    nb = s * RPS
    pltpu.sync_copy(zeros_hbm.at[pl.ds(nb, RPS)], acc.at[pl.ds(nb, RPS)])
    lane = lax.iota(jnp.int32, 16)
    base_row = jnp.where(lane == 0, 1.0, 0.0).astype(jnp.float32)
    e1 = jnp.where(lane == 1, 1.0, 0.0).astype(jnp.float32)
    plsc.subcore_barrier()

    pltpu.sync_copy(ei_hbm.at[1, wid], dstbuf)
    pltpu.sync_copy(ew_hbm.at[wid], ewbuf)

    def stage(ob, carry):
        def body(j, c2):
            for g in range(CHUNK // 16):
                ewv = ewbuf[ob, j, pl.ds(g * 16, 16)]
                for k in range(16):
                    msg[g * 16 + k] = base_row + e1 * ewv[k]
            pltpu.sync_copy(msg, acc.at[dstbuf.at[ob, j]], add=True)
            return c2

        lax.fori_loop(0, SB, body, 0)
        return carry

    lax.fori_loop(0, NSTAGE, stage, 0)
    plsc.subcore_barrier()
    pltpu.sync_copy(acc.at[pl.ds(nb, RPS)], degp_hbm.at[c, pl.ds(nb, RPS)])


def _sc_deg(ei5d, ew4d, z16):
    fn = functools.partial(
        pl.kernel,
        out_type=jax.ShapeDtypeStruct((NC, NP, 16), jnp.float32),
        mesh=_mesh(),
        compiler_params=pltpu.CompilerParams(use_tc_tiling_on_sc=False),
        scratch_types=[
            pltpu.VMEM((NSTAGE, SB, CHUNK), jnp.int32),
            pltpu.VMEM((NSTAGE, SB, CHUNK), jnp.float32),
            pltpu.VMEM((CHUNK, 16), jnp.float32),
            pltpu.VMEM_SHARED((NP, 16), jnp.float32),
        ],
    )(_sc_deg_body)
    return fn(ei5d, ew4d, z16)


def _sc_agg1_body(hp_hbm, ei_hbm, zeros_hbm, agg_hbm,
                  srcbuf, dstbuf, rows0, rows1, acc, sem0, sem1):
    c = lax.axis_index("c")
    s = lax.axis_index("s")
    wid = c * NS + s
    nb = s * RPS
    pltpu.sync_copy(zeros_hbm.at[pl.ds(nb, RPS)], acc.at[pl.ds(nb, RPS)])
    plsc.subcore_barrier()

    def stage(ob, carry):
        pltpu.sync_copy(ei_hbm.at[0, wid, ob], srcbuf)
        pltpu.sync_copy(ei_hbm.at[1, wid, ob], dstbuf)
        pltpu.async_copy(hp_hbm.at[srcbuf.at[0]], rows0, sem0)

        def body(t, c2):
            j = t * 2
            pltpu.async_copy(hp_hbm.at[srcbuf.at[j + 1]], rows1, sem1)
            pltpu.make_async_copy(hp_hbm.at[srcbuf.at[j]], rows0, sem0).wait()
            pltpu.sync_copy(rows0, acc.at[dstbuf.at[j]], add=True)
            pltpu.async_copy(hp_hbm.at[srcbuf.at[j + 2]], rows0, sem0)
            pltpu.make_async_copy(hp_hbm.at[srcbuf.at[j + 1]], rows1, sem1).wait()
            pltpu.sync_copy(rows1, acc.at[dstbuf.at[j + 1]], add=True)
            return c2

        lax.fori_loop(0, (SB - 1) // 2, body, 0)
        pltpu.make_async_copy(hp_hbm.at[srcbuf.at[SB - 1]], rows0, sem0).wait()
        pltpu.sync_copy(rows0, acc.at[dstbuf.at[SB - 1]], add=True)
        return carry

    lax.fori_loop(0, NSTAGE, stage, 0)
    plsc.subcore_barrier()
    pltpu.sync_copy(acc.at[pl.ds(nb, RPS)], agg_hbm.at[c, pl.ds(nb, RPS)])


def _sc_agg1(hp, ei5d, z128):
    fn = functools.partial(
        pl.kernel,
        out_type=jax.ShapeDtypeStruct((NC, NP, D_HID), jnp.float32),
        mesh=_mesh(),
        compiler_params=pltpu.CompilerParams(use_tc_tiling_on_sc=False),
        scratch_types=[
            pltpu.VMEM((SB, CHUNK), jnp.int32),
            pltpu.VMEM((SB, CHUNK), jnp.int32),
            pltpu.VMEM((CHUNK, D_HID), jnp.float32),
            pltpu.VMEM((CHUNK, D_HID), jnp.float32),
            pltpu.VMEM_SHARED((NP, D_HID), jnp.float32),
            pltpu.SemaphoreType.DMA,
            pltpu.SemaphoreType.DMA,
        ],
    )(_sc_agg1_body)
    return fn(hp, ei5d, z128)


def _sc_agg2_body(zp_hbm, ei_hbm, ew_hbm, zeros_hbm, agg_hbm,
                  srcbuf, dstbuf, ewbuf, rows0, rows1, acc, sem0, sem1):
    c = lax.axis_index("c")
    s = lax.axis_index("s")
    wid = c * NS + s
    nb = s * RPS
    pltpu.sync_copy(zeros_hbm.at[pl.ds(nb, RPS)], acc.at[pl.ds(nb, RPS)])
    plsc.subcore_barrier()

    pltpu.sync_copy(ei_hbm.at[0, wid], srcbuf)
    pltpu.sync_copy(ei_hbm.at[1, wid], dstbuf)
    pltpu.sync_copy(ew_hbm.at[wid], ewbuf)

    def scale(rows_v, ob, j):
        for g in range(CHUNK // 16):
            ewv = ewbuf[ob, j, pl.ds(g * 16, 16)]
            for k in range(16):
                r = g * 16 + k
                rows_v[r] = rows_v[r] * ewv[k]

    def stage(ob, carry):
        pltpu.async_copy(zp_hbm.at[srcbuf.at[ob, 0]], rows0, sem0)

        def body(t, c2):
            j = t * 2
            pltpu.async_copy(zp_hbm.at[srcbuf.at[ob, j + 1]], rows1, sem1)
            pltpu.make_async_copy(zp_hbm.at[srcbuf.at[ob, j]], rows0, sem0).wait()
            scale(rows0, ob, j)
            pltpu.sync_copy(rows0, acc.at[dstbuf.at[ob, j]], add=True)
            pltpu.async_copy(zp_hbm.at[srcbuf.at[ob, j + 2]], rows0, sem0)
            pltpu.make_async_copy(zp_hbm.at[srcbuf.at[ob, j + 1]], rows1, sem1).wait()
            scale(rows1, ob, j + 1)
            pltpu.sync_copy(rows1, acc.at[dstbuf.at[ob, j + 1]], add=True)
            return c2

        lax.fori_loop(0, (SB - 1) // 2, body, 0)
        pltpu.make_async_copy(zp_hbm.at[srcbuf.at[ob, SB - 1]], rows0, sem0).wait()
        scale(rows0, ob, SB - 1)
        pltpu.sync_copy(rows0, acc.at[dstbuf.at[ob, SB - 1]], add=True)
        return carry

    lax.fori_loop(0, NSTAGE, stage, 0)
    plsc.subcore_barrier()
    pltpu.sync_copy(acc.at[pl.ds(nb, RPS)], agg_hbm.at[c, pl.ds(nb, RPS)])


def _sc_agg2(zp, ei5d, ew4d, z16):
    fn = functools.partial(
        pl.kernel,
        out_type=jax.ShapeDtypeStruct((NC, NP, D_OUT), jnp.float32),
        mesh=_mesh(),
        compiler_params=pltpu.CompilerParams(use_tc_tiling_on_sc=False),
        scratch_types=[
            pltpu.VMEM((NSTAGE, SB, CHUNK), jnp.int32),
            pltpu.VMEM((NSTAGE, SB, CHUNK), jnp.int32),
            pltpu.VMEM((NSTAGE, SB, CHUNK), jnp.float32),
            pltpu.VMEM((CHUNK, D_OUT), jnp.float32),
            pltpu.VMEM((CHUNK, D_OUT), jnp.float32),
            pltpu.VMEM_SHARED((NP, D_OUT), jnp.float32),
            pltpu.SemaphoreType.DMA,
            pltpu.SemaphoreType.DMA,
        ],
    )(_sc_agg2_body)
    return fn(zp, ei5d, ew4d, z16)


def _tc_h_body(x_ref, w_ref, out_ref):
    out_ref[...] = jnp.dot(x_ref[...], w_ref[...],
                           preferred_element_type=jnp.float32)


def _tc_h(x, w1):
    return pl.pallas_call(
        _tc_h_body,
        out_shape=jax.ShapeDtypeStruct((N, D_HID), jnp.float32),
    )(x, w1)


def _tc_hp_body(h_ref, degp_ref, out_ref):
    deg1 = degp_ref[0, 0:N, 0:1] + degp_ref[1, 0:N, 0:1] + 1.0
    out_ref[...] = h_ref[...] * lax.rsqrt(deg1)


def _tc_hp(h, degp):
    return pl.pallas_call(
        _tc_hp_body,
        out_shape=jax.ShapeDtypeStruct((N, D_HID), jnp.float32),
    )(h, degp)


def _tc_mid_body(agg_ref, hp_ref, degp_ref, w2_ref, b1_ref, out_ref):
    d1 = lax.rsqrt(degp_ref[0, 0:N, 0:1] + degp_ref[1, 0:N, 0:1] + 1.0)
    d2 = lax.rsqrt(degp_ref[0, 0:N, 1:2] + degp_ref[1, 0:N, 1:2] + 1.0)
    tot = agg_ref[0, 0:N, :] + agg_ref[1, 0:N, :] + hp_ref[...]
    x2 = jnp.maximum(tot * d1 + b1_ref[...][None, :], 0.0)
    z = jnp.dot(x2, w2_ref[...], preferred_element_type=jnp.float32)
    out_ref[...] = z * d2


def _tc_mid(agg1, hp, degp, w2, b1):
    return pl.pallas_call(
        _tc_mid_body,
        out_shape=jax.ShapeDtypeStruct((N, D_OUT), jnp.float32),
    )(agg1, hp, degp, w2, b1)


def _tc_out_body(agg_ref, zp_ref, degp_ref, b2_ref, out_ref):
    d2 = lax.rsqrt(degp_ref[0, 0:N, 1:2] + degp_ref[1, 0:N, 1:2] + 1.0)
    y = (agg_ref[0, 0:N, :] + agg_ref[1, 0:N, :] + zp_ref[...]) * d2
    y = y + b2_ref[...][None, :]
    m = jnp.max(y, axis=1, keepdims=True)
    ex = jnp.exp(y - m)
    out_ref[...] = y - m - jnp.log(jnp.sum(ex, axis=1, keepdims=True))


def _tc_out(agg2, zp, degp, b2):
    return pl.pallas_call(
        _tc_out_body,
        out_shape=jax.ShapeDtypeStruct((N, D_OUT), jnp.float32),
    )(agg2, zp, degp, b2)


def kernel(features, edge_index, edge_weight, W1, b1, W2, b2):
    ei5d = edge_index.reshape(2, NW, NSTAGE, SB, CHUNK)
    ew4d = edge_weight.reshape(NW, NSTAGE, SB, CHUNK)
    z16 = jnp.zeros((NP, 16), jnp.float32)
    z128 = jnp.zeros((NP, D_HID), jnp.float32)

    h = _tc_h(features, W1)
    degp = _sc_deg(ei5d, ew4d, z16)
    hp = _tc_hp(h, degp)
    agg1 = _sc_agg1(hp, ei5d, z128)
    zp = _tc_mid(agg1, hp, degp, W2, b1)
    agg2 = _sc_agg2(zp, ei5d, ew4d, z16)
    return _tc_out(agg2, zp, degp, b2)
